# baseline (speedup 1.0000x reference)
import functools
import math

import jax
import jax.numpy as jnp
from jax import lax
from jax.experimental import pallas as pl
from jax.experimental.pallas import tpu as pltpu
from jax.experimental.pallas import tpu_sc as plsc

D_MODEL = 2048
B, S = 4, 2048
NTOK = B * S
SCALE = math.sqrt(D_MODEL)

NC, NS, L = 2, 16, 16
NW = NC * NS
TPW = NTOK // NW
K = 8
CHUNKS = TPW // K


def _twin_emb_body(ut, up, us, vt, vp, vs, t1, p1, s1, t2, p2, s2,
                   out_u, out_v, itok, ipos, iseg,
                   a0, b0, a1, b1, ob0, ob1, seg1v, seg2v, sg0, sg1, st0, st1):
    wid = lax.axis_index("s") * NC + lax.axis_index("c")
    base = wid * TPW
    crow = wid * CHUNKS
    gsets = ((a0, b0, sg0), (a1, b1, sg1))
    obufs = (ob0, ob1)
    ssts = (st0, st1)

    pltpu.sync_copy(s1, seg1v)
    pltpu.sync_copy(s2, seg2v)

    def fire(tabs, c, s):
        a, b, sem = gsets[s]
        pltpu.async_copy(tabs[0].at[itok.at[c]], a, sem)
        pltpu.async_copy(tabs[1].at[ipos.at[c]], b, sem)

    def drain_g(tabs, c, s):
        a, b, sem = gsets[s]
        pltpu.make_async_copy(tabs[0].at[itok.at[c]], a, sem).wait()
        pltpu.make_async_copy(tabs[1].at[ipos.at[c]], b, sem).wait()

    def fire_store(out, c, s):
        row0 = pl.multiple_of(base + c * K, K)
        pltpu.async_copy(obufs[s], out.at[pl.ds(row0, K)], ssts[s])

    def drain_store(out, s):
        pltpu.make_async_copy(obufs[s], out.at[pl.ds(0, K)], ssts[s]).wait()

    def compute(s, segtab, segv):
        a, b, _ = gsets[s]
        obuf = obufs[s]
        frs = [segv[s * K + r].astype(jnp.float32) for r in range(K)]

        @plsc.parallel_loop(0, D_MODEL, step=L, unroll=4)
        def col(i):
            sl = pl.ds(pl.multiple_of(i, L), L)
            c0 = segtab[0, sl]
            d = segtab[1, sl] - c0
            for r in range(K):
                c = c0 + frs[r] * d
                obuf[r, sl] = (a[r, sl] + b[r, sl] + c) * SCALE

    towers = (
        (ut, up, us, (t1, p1), seg1v, out_u),
        (vt, vp, vs, (t2, p2), seg2v, out_v),
    )

    for t, (tix, pix, six, tabs, segtab, out) in enumerate(towers):
        pltpu.sync_copy(tix.at[pl.ds(crow, CHUNKS)], itok)
        pltpu.sync_copy(pix.at[pl.ds(crow, CHUNKS)], ipos)
        pltpu.sync_copy(six.at[wid], iseg)
        fire(tabs, 0, 0)
        fire(tabs, 1, 1)

        def pair(p, carry, tabs=tabs, segtab=segtab, out=out, t=t):
            segv = iseg[pl.ds(pl.multiple_of(p * 2 * K, 2 * K), 2 * K)]
            for s in (0, 1):
                cc = 2 * p + s
                if t == 0:
                    @pl.when(p > 0)
                    def _(s=s):
                        drain_store(out, s)
                else:
                    drain_store(out, s)
                drain_g(tabs, cc, s)
                compute(s, segtab, segv)
                fire_store(out, cc, s)

                @pl.when(cc + 2 < CHUNKS)
                def _(cc=cc, s=s):
                    fire(tabs, cc + 2, s)
            return carry

        lax.fori_loop(0, CHUNKS // 2, pair, 0)

    drain_store(out_v, 0)
    drain_store(out_v, 1)


@jax.jit
def _twin_emb(ut, up, us, vt, vp, vs, t1, p1, s1, t2, p2, s2):
    mesh = plsc.VectorSubcoreMesh(core_axis_name="c", subcore_axis_name="s")
    f = functools.partial(
        pl.kernel,
        out_type=(
            jax.ShapeDtypeStruct((NTOK, D_MODEL), jnp.float32),
            jax.ShapeDtypeStruct((NTOK, D_MODEL), jnp.float32),
        ),
        mesh=mesh,
        scratch_types=[
            pltpu.VMEM((CHUNKS, K), jnp.int32),
            pltpu.VMEM((CHUNKS, K), jnp.int32),
            pltpu.VMEM((TPW,), jnp.int32),
            pltpu.VMEM((K, D_MODEL), jnp.float32),
            pltpu.VMEM((K, D_MODEL), jnp.float32),
            pltpu.VMEM((K, D_MODEL), jnp.float32),
            pltpu.VMEM((K, D_MODEL), jnp.float32),
            pltpu.VMEM((K, D_MODEL), jnp.float32),
            pltpu.VMEM((K, D_MODEL), jnp.float32),
            pltpu.VMEM((2, D_MODEL), jnp.float32),
            pltpu.VMEM((2, D_MODEL), jnp.float32),
            pltpu.SemaphoreType.DMA,
            pltpu.SemaphoreType.DMA,
            pltpu.SemaphoreType.DMA,
            pltpu.SemaphoreType.DMA,
        ],
    )(_twin_emb_body)
    return f(ut, up, us, vt, vp, vs, t1, p1, s1, t2, p2, s2)


def kernel(u_tok, u_pos, u_seg, v_tok, v_pos, v_seg,
           tok1, pos1, seg1, tok2, pos2, seg2):
    def prep(ix):
        return ix.reshape(NTOK // K, K).astype(jnp.int32)

    def prep_seg(ix):
        return ix.reshape(NW, TPW).astype(jnp.int32)

    out_u, out_v = _twin_emb(
        prep(u_tok), prep(u_pos), prep_seg(u_seg),
        prep(v_tok), prep(v_pos), prep_seg(v_seg),
        tok1, pos1, seg1, tok2, pos2, seg2)
    return (out_u.reshape(B, S, D_MODEL), out_v.reshape(B, S, D_MODEL))


# speedup vs baseline: 1.0205x; 1.0205x over previous
import functools
import math

import jax
import jax.numpy as jnp
from jax import lax
from jax.experimental import pallas as pl
from jax.experimental.pallas import tpu as pltpu
from jax.experimental.pallas import tpu_sc as plsc

D_MODEL = 2048
B, S = 4, 2048
NTOK = B * S
SCALE = math.sqrt(D_MODEL)

NC, NS, L = 2, 16, 16
NW = NC * NS
TPW = NTOK // NW
K = 8
CHUNKS = TPW // K


def _twin_emb_body(ut, up, us, vt, vp, vs, t1, p1, s1, t2, p2, s2,
                   out_u, out_v, itok, ipos, iseg,
                   a0, b0, a1, b1, ob0, ob1, seg1v, seg2v, sg0, sg1, st0, st1):
    wid = lax.axis_index("s") * NC + lax.axis_index("c")
    base = wid * TPW
    crow = wid * CHUNKS
    gsets = ((a0, b0, sg0), (a1, b1, sg1))
    obufs = (ob0, ob1)
    ssts = (st0, st1)

    pltpu.sync_copy(s1, seg1v)
    pltpu.sync_copy(s2, seg2v)

    def fire(tabs, c, s):
        a, b, sem = gsets[s]
        pltpu.async_copy(tabs[0].at[itok.at[c]], a, sem)
        pltpu.async_copy(tabs[1].at[ipos.at[c]], b, sem)

    def drain_g(tabs, c, s):
        a, b, sem = gsets[s]
        pltpu.make_async_copy(tabs[0].at[itok.at[c]], a, sem).wait()
        pltpu.make_async_copy(tabs[1].at[ipos.at[c]], b, sem).wait()

    def fire_store(out, c, s):
        row0 = pl.multiple_of(base + c * K, K)
        pltpu.async_copy(obufs[s], out.at[pl.ds(row0, K)], ssts[s])

    def drain_store(out, s):
        pltpu.make_async_copy(obufs[s], out.at[pl.ds(0, K)], ssts[s]).wait()

    def compute(s, segtab, segv):
        a, b, _ = gsets[s]
        obuf = obufs[s]
        frs = [segv[s * K + r].astype(jnp.float32) for r in range(K)]

        @plsc.parallel_loop(0, D_MODEL, step=L, unroll=2)
        def col(i):
            sl = pl.ds(pl.multiple_of(i, L), L)
            c0 = segtab[0, sl]
            d = segtab[1, sl] - c0
            for r in range(K):
                c = c0 + frs[r] * d
                obuf[r, sl] = (a[r, sl] + b[r, sl] + c) * SCALE

    tabs_u, tabs_v = (t1, p1), (t2, p2)
    towers = (
        (tabs_u, seg1v, out_u),
        (tabs_v, seg2v, out_v),
    )

    pltpu.sync_copy(ut.at[pl.ds(crow, CHUNKS)], itok.at[pl.ds(0, CHUNKS)])
    pltpu.sync_copy(up.at[pl.ds(crow, CHUNKS)], ipos.at[pl.ds(0, CHUNKS)])
    fire(tabs_u, 0, 0)
    fire(tabs_u, 1, 1)
    pltpu.sync_copy(vt.at[pl.ds(crow, CHUNKS)], itok.at[pl.ds(CHUNKS, CHUNKS)])
    pltpu.sync_copy(vp.at[pl.ds(crow, CHUNKS)], ipos.at[pl.ds(CHUNKS, CHUNKS)])
    pltpu.sync_copy(us.at[wid], iseg.at[pl.ds(0, TPW)])
    pltpu.sync_copy(vs.at[wid], iseg.at[pl.ds(TPW, TPW)])

    for t, (tabs, segtab, out) in enumerate(towers):
        def pair(p, carry, tabs=tabs, segtab=segtab, out=out, t=t):
            segv = iseg[pl.ds(pl.multiple_of(t * TPW + p * 2 * K, 2 * K),
                              2 * K)]
            for s in (0, 1):
                cc = 2 * p + s
                gc = cc + t * CHUNKS
                if t == 0:
                    @pl.when(p > 0)
                    def _(s=s):
                        drain_store(out, s)
                else:
                    drain_store(out, s)
                drain_g(tabs, gc, s)
                compute(s, segtab, segv)
                fire_store(out, cc, s)

                @pl.when(cc + 2 < CHUNKS)
                def _(gc=gc, s=s):
                    fire(tabs, gc + 2, s)

                if t == 0:
                    @pl.when(cc + 2 >= CHUNKS)
                    def _(gc=gc, s=s):
                        fire(tabs_v, gc + 2, s)
            return carry

        lax.fori_loop(0, CHUNKS // 2, pair, 0)

    drain_store(out_v, 0)
    drain_store(out_v, 1)


@jax.jit
def _twin_emb(ut, up, us, vt, vp, vs, t1, p1, s1, t2, p2, s2):
    mesh = plsc.VectorSubcoreMesh(core_axis_name="c", subcore_axis_name="s")
    f = functools.partial(
        pl.kernel,
        out_type=(
            jax.ShapeDtypeStruct((NTOK, D_MODEL), jnp.float32),
            jax.ShapeDtypeStruct((NTOK, D_MODEL), jnp.float32),
        ),
        mesh=mesh,
        scratch_types=[
            pltpu.VMEM((2 * CHUNKS, K), jnp.int32),
            pltpu.VMEM((2 * CHUNKS, K), jnp.int32),
            pltpu.VMEM((2 * TPW,), jnp.int32),
            pltpu.VMEM((K, D_MODEL), jnp.float32),
            pltpu.VMEM((K, D_MODEL), jnp.float32),
            pltpu.VMEM((K, D_MODEL), jnp.float32),
            pltpu.VMEM((K, D_MODEL), jnp.float32),
            pltpu.VMEM((K, D_MODEL), jnp.float32),
            pltpu.VMEM((K, D_MODEL), jnp.float32),
            pltpu.VMEM((2, D_MODEL), jnp.float32),
            pltpu.VMEM((2, D_MODEL), jnp.float32),
            pltpu.SemaphoreType.DMA,
            pltpu.SemaphoreType.DMA,
            pltpu.SemaphoreType.DMA,
            pltpu.SemaphoreType.DMA,
        ],
    )(_twin_emb_body)
    return f(ut, up, us, vt, vp, vs, t1, p1, s1, t2, p2, s2)


def kernel(u_tok, u_pos, u_seg, v_tok, v_pos, v_seg,
           tok1, pos1, seg1, tok2, pos2, seg2):
    def prep(ix):
        return ix.reshape(NTOK // K, K).astype(jnp.int32)

    def prep_seg(ix):
        return ix.reshape(NW, TPW).astype(jnp.int32)

    out_u, out_v = _twin_emb(
        prep(u_tok), prep(u_pos), prep_seg(u_seg),
        prep(v_tok), prep(v_pos), prep_seg(v_seg),
        tok1, pos1, seg1, tok2, pos2, seg2)
    return (out_u.reshape(B, S, D_MODEL), out_v.reshape(B, S, D_MODEL))
